# baseline (speedup 1.0000x reference)
import functools

import jax
import jax.numpy as jnp
from jax import lax
from jax.experimental import pallas as pl
from jax.experimental.pallas import tpu as pltpu
from jax.experimental.pallas import tpu_sc as plsc

_NC = 2
_NS = 16
_NW = _NC * _NS
_CHUNK = 80
_ACCD = 1280
_NP = 4


@functools.lru_cache(maxsize=None)
def _make_sc_agg(n, d, nch):
    rpt = _ACCD // _NS
    zrows = rpt
    assert rpt % 8 == 0 and rpt % zrows == 0 and d % 128 == 0

    mesh = plsc.VectorSubcoreMesh(
        core_axis_name="c", subcore_axis_name="s",
        num_cores=_NC, num_subcores=_NS)

    def body(x_hbm, src_hbm, dstl_hbm, out_hbm,
             src_v, dst_v, rows_v, zrow_v, acc_sh, sem):
        cid = lax.axis_index("c")
        sid = lax.axis_index("s")
        wid = cid * _NS + sid

        def zfill(i, carry):
            zrow_v[i // 8, pl.ds((i % 8) * 16, 16)] = jnp.zeros((16,),
                                                               jnp.float32)
            return carry

        lax.fori_loop(0, zrows * d // 16, zfill, 0)
        pltpu.sync_copy(src_hbm.at[wid], src_v)

        for p in range(_NP):
            for j in range(rpt // zrows):
                pltpu.sync_copy(
                    zrow_v, acc_sh.at[pl.ds(sid * rpt + j * zrows, zrows), :])
            pltpu.sync_copy(dstl_hbm.at[p, wid], dst_v)
            plsc.subcore_barrier()

            def edge_chunk(c, carry):
                pltpu.async_copy(x_hbm.at[src_v.at[c]], rows_v, sem).wait()
                pltpu.sync_copy(rows_v, acc_sh.at[dst_v.at[c]], add=True)
                return carry

            lax.fori_loop(0, nch, edge_chunk, 0)
            plsc.subcore_barrier()

            pltpu.sync_copy(acc_sh.at[pl.ds(sid * rpt, rpt), :],
                            out_hbm.at[p, cid, pl.ds(sid * rpt, rpt), :])

    return pl.kernel(
        body,
        out_type=jax.ShapeDtypeStruct((_NP, _NC, _ACCD, d), jnp.float32),
        mesh=mesh,
        scratch_types=[
            pltpu.VMEM((nch, _CHUNK), jnp.int32),
            pltpu.VMEM((nch, _CHUNK), jnp.int32),
            pltpu.VMEM((_CHUNK, d), jnp.float32),
            pltpu.VMEM((zrows, d), jnp.float32),
            pltpu.VMEM_SHARED((_ACCD + 8, d), jnp.float32),
            pltpu.SemaphoreType.DMA,
        ])


def _tc_body(x_ref, agg_ref, deg_ref, wx_ref, wa_ref, b_ref, o_ref):
    deg = jnp.maximum(deg_ref[:, 0:1], 1.0)
    agg = agg_ref[...] / deg
    h = (jnp.dot(x_ref[...], wx_ref[...], preferred_element_type=jnp.float32)
         + jnp.dot(agg, wa_ref[...], preferred_element_type=jnp.float32)
         + b_ref[...])
    ssq = jnp.sum(h * h, axis=-1, keepdims=True)
    h = h * lax.rsqrt(jnp.maximum(ssq, 1e-12))
    o_ref[...] = jnp.maximum(h, 0.0)


@functools.lru_cache(maxsize=None)
def _make_tc_layer(n, d, hout, blk):
    row = lambda w: pl.BlockSpec((blk, w), lambda i: (i, 0))
    full = lambda r, c: pl.BlockSpec((r, c), lambda i: (0, 0))
    return pl.pallas_call(
        _tc_body,
        grid=(n // blk,),
        in_specs=[row(d), row(d), row(16),
                  full(d, hout), full(d, hout), full(1, hout)],
        out_specs=row(hout),
        out_shape=jax.ShapeDtypeStruct((n, hout), jnp.float32),
    )


def kernel(embeddings, edge_index, W0, b0, W1, b1, W2, b2, W3, b3):
    n, d = embeddings.shape
    e = edge_index.shape[1]
    assert e % (_NS * _CHUNK) == 0 and n % 1000 == 0
    assert n <= _NP * _NC * _ACCD
    nch = e // (_NS * _CHUNK)

    ei = edge_index.astype(jnp.int32)
    src16 = ei[0].reshape(_NS, nch, _CHUNK)
    src = jnp.broadcast_to(src16[None], (_NC, _NS, nch, _CHUNK)
                           ).reshape(_NW, nch, _CHUNK)
    dst16 = ei[1].reshape(_NS, nch, _CHUNK)
    seg = jnp.arange(_NP * _NC, dtype=jnp.int32) * _ACCD
    local = dst16[None] - seg[:, None, None, None]
    dstl = jnp.where((local >= 0) & (local < _ACCD), local, _ACCD
                     ).reshape(_NP, _NW, nch, _CHUNK)

    sc_agg = _make_sc_agg(n, d, nch)

    degf = sc_agg(jnp.ones((n, d), jnp.float32), src, dstl)
    deg2d = degf.reshape(_NP * _NC * _ACCD, d)[:, :16]

    h = embeddings
    for w, b in ((W0, b0), (W1, b1), (W2, b2), (W3, b3)):
        hout = w.shape[1]
        agg = sc_agg(h, src, dstl).reshape(_NP * _NC * _ACCD, d)
        tc = _make_tc_layer(n, d, hout, 1000)
        h = tc(h, agg, deg2d, w[:d], w[d:], b.reshape(1, hout))
    return h


# speedup vs baseline: 1.1160x; 1.1160x over previous
import functools

import jax
import jax.numpy as jnp
from jax import lax
from jax.experimental import pallas as pl
from jax.experimental.pallas import tpu as pltpu
from jax.experimental.pallas import tpu_sc as plsc

_NC = 2
_NS = 16
_NW = _NC * _NS
_CHUNK = 80
_ACCD = 1280
_NP = 4


@functools.lru_cache(maxsize=None)
def _make_sc_agg(n, d, nch):
    rpt = _ACCD // _NS
    zrows = rpt
    assert rpt % 8 == 0 and rpt % zrows == 0 and d % 128 == 0

    mesh = plsc.VectorSubcoreMesh(
        core_axis_name="c", subcore_axis_name="s",
        num_cores=_NC, num_subcores=_NS)

    assert nch % 2 == 0

    def body(x_hbm, src_hbm, dstl_hbm, out_hbm,
             src_v, dst_v, rows_a, rows_b, zrow_v, acc_sh, sem_a, sem_b):
        cid = lax.axis_index("c")
        sid = lax.axis_index("s")
        wid = cid * _NS + sid

        def zfill(i, carry):
            zrow_v[i // 8, pl.ds((i % 8) * 16, 16)] = jnp.zeros((16,),
                                                               jnp.float32)
            return carry

        lax.fori_loop(0, zrows * d // 16, zfill, 0)
        pltpu.sync_copy(src_hbm.at[wid], src_v)

        def gather(c, buf, sem):
            return pltpu.make_async_copy(x_hbm.at[src_v.at[c]], buf, sem)

        for p in range(_NP):
            for j in range(rpt // zrows):
                pltpu.sync_copy(
                    zrow_v, acc_sh.at[pl.ds(sid * rpt + j * zrows, zrows), :])
            pltpu.sync_copy(dstl_hbm.at[p, wid], dst_v)
            plsc.subcore_barrier()

            gather(0, rows_a, sem_a).start()

            def edge_pair(i, carry):
                c0 = 2 * i
                gather(c0 + 1, rows_b, sem_b).start()
                gather(c0, rows_a, sem_a).wait()
                pltpu.sync_copy(rows_a, acc_sh.at[dst_v.at[c0]], add=True)
                gather(lax.rem(c0 + 2, nch), rows_a, sem_a).start()
                gather(c0 + 1, rows_b, sem_b).wait()
                pltpu.sync_copy(rows_b, acc_sh.at[dst_v.at[c0 + 1]], add=True)
                return carry

            lax.fori_loop(0, nch // 2, edge_pair, 0)
            gather(0, rows_a, sem_a).wait()
            plsc.subcore_barrier()

            pltpu.sync_copy(acc_sh.at[pl.ds(sid * rpt, rpt), :],
                            out_hbm.at[p, cid, pl.ds(sid * rpt, rpt), :])

    return pl.kernel(
        body,
        out_type=jax.ShapeDtypeStruct((_NP, _NC, _ACCD, d), jnp.float32),
        mesh=mesh,
        scratch_types=[
            pltpu.VMEM((nch, _CHUNK), jnp.int32),
            pltpu.VMEM((nch, _CHUNK), jnp.int32),
            pltpu.VMEM((_CHUNK, d), jnp.float32),
            pltpu.VMEM((_CHUNK, d), jnp.float32),
            pltpu.VMEM((zrows, d), jnp.float32),
            pltpu.VMEM_SHARED((_ACCD + 8, d), jnp.float32),
            pltpu.SemaphoreType.DMA,
            pltpu.SemaphoreType.DMA,
        ])


def _tc_body(x_ref, agg_ref, deg_ref, wx_ref, wa_ref, b_ref, o_ref):
    deg = jnp.maximum(deg_ref[:, 0:1], 1.0)
    agg = agg_ref[...] / deg
    h = (jnp.dot(x_ref[...], wx_ref[...], preferred_element_type=jnp.float32)
         + jnp.dot(agg, wa_ref[...], preferred_element_type=jnp.float32)
         + b_ref[...])
    ssq = jnp.sum(h * h, axis=-1, keepdims=True)
    h = h * lax.rsqrt(jnp.maximum(ssq, 1e-12))
    o_ref[...] = jnp.maximum(h, 0.0)


@functools.lru_cache(maxsize=None)
def _make_tc_layer(n, d, hout, blk):
    row = lambda w: pl.BlockSpec((blk, w), lambda i: (i, 0))
    full = lambda r, c: pl.BlockSpec((r, c), lambda i: (0, 0))
    return pl.pallas_call(
        _tc_body,
        grid=(n // blk,),
        in_specs=[row(d), row(d), row(16),
                  full(d, hout), full(d, hout), full(1, hout)],
        out_specs=row(hout),
        out_shape=jax.ShapeDtypeStruct((n, hout), jnp.float32),
    )


def kernel(embeddings, edge_index, W0, b0, W1, b1, W2, b2, W3, b3):
    n, d = embeddings.shape
    e = edge_index.shape[1]
    assert e % (_NS * _CHUNK) == 0 and n % 1000 == 0
    assert n <= _NP * _NC * _ACCD
    nch = e // (_NS * _CHUNK)

    ei = edge_index.astype(jnp.int32)
    src16 = ei[0].reshape(_NS, nch, _CHUNK)
    src = jnp.broadcast_to(src16[None], (_NC, _NS, nch, _CHUNK)
                           ).reshape(_NW, nch, _CHUNK)
    dst16 = ei[1].reshape(_NS, nch, _CHUNK)
    seg = jnp.arange(_NP * _NC, dtype=jnp.int32) * _ACCD
    local = dst16[None] - seg[:, None, None, None]
    dstl = jnp.where((local >= 0) & (local < _ACCD), local, _ACCD
                     ).reshape(_NP, _NW, nch, _CHUNK)

    sc_agg = _make_sc_agg(n, d, nch)

    degf = sc_agg(jnp.ones((n, d), jnp.float32), src, dstl)
    deg2d = degf.reshape(_NP * _NC * _ACCD, d)[:, :16]

    h = embeddings
    for w, b in ((W0, b0), (W1, b1), (W2, b2), (W3, b3)):
        hout = w.shape[1]
        agg = sc_agg(h, src, dstl).reshape(_NP * _NC * _ACCD, d)
        tc = _make_tc_layer(n, d, hout, 1000)
        h = tc(h, agg, deg2d, w[:d], w[d:], b.reshape(1, hout))
    return h
